# baseline (speedup 1.0000x reference)
import functools

import jax
import jax.numpy as jnp
from jax import lax
from jax.experimental import pallas as pl
from jax.experimental.pallas import tpu as pltpu
from jax.experimental.pallas import tpu_sc as plsc

_D = 64
_NC, _NS = 2, 16
_NW = _NC * _NS
_V = 1000000
_FULL = _V // 128
_BPT = _FULL // _NW
_REM = _FULL - _BPT * _NW
_TAILN = _V - _FULL * 128

_N, _S = 4096, 200
_B = _N * _S
_JBLK = _B // 128
_JPT = _JBLK // _NW


def _mesh():
    return plsc.VectorSubcoreMesh(core_axis_name="c", subcore_axis_name="s")


def _iota16():
    return lax.iota(jnp.int32, 16)


def _splat(x):
    return jnp.full((16,), x, jnp.int32)


@jax.jit
def _detranspose(table_t):
    @functools.partial(
        pl.kernel,
        mesh=_mesh(),
        out_type=jax.ShapeDtypeStruct((_V // 16, 8, 128), jnp.float32),
        scratch_types=[
            pltpu.VMEM((2, 8, 8, 128), jnp.float32),
            pltpu.VMEM((2, 8, 8, 128), jnp.float32),
            pltpu.SemaphoreType.DMA,
            pltpu.SemaphoreType.DMA,
            pltpu.SemaphoreType.DMA,
            pltpu.SemaphoreType.DMA,
        ],
        compiler_params=pltpu.CompilerParams(use_tc_tiling_on_sc=True, needs_layout_passes=False),
    )
    def ka(tt_hbm, out_hbm, inb, outb, gi0, gi1, wo0, wo1):
        wid = lax.axis_index("s") * _NC + lax.axis_index("c")
        gsem = (gi0, gi1)
        wsem = (wo0, wo1)
        iot = _iota16()
        dtr = [(iot + 16 * g) >> 3 for g in range(4)]
        ddr = [(iot + 16 * g) & 7 for g in range(4)]

        def blk_of(i):
            return i * _NW + wid

        def load(i, b):
            blk = blk_of(i)
            return [
                pltpu.make_async_copy(
                    tt_hbm.at[pl.ds(tr * 8, 8), pl.ds(blk * 128, 128)],
                    inb.at[b, tr], gsem[b])
                for tr in range(8)
            ]

        def store(i, b):
            blk = blk_of(i)
            return pltpu.make_async_copy(
                outb.at[b], out_hbm.at[pl.ds(blk * 8, 8)], wsem[b])

        def transpose(b):
            def nbody(n4, _):
                for u in range(4):
                    n = n4 * 4 + u
                    nsp = _splat(n)
                    flat0 = n * _D
                    for g in range(4):
                        v = plsc.load_gather(inb.at[b], [dtr[g], ddr[g], nsp])
                        f = flat0 + 16 * g
                        plsc.store_scatter(
                            outb.at[b],
                            [_splat(f // 1024), _splat((f // 128) % 8),
                             _splat(f % 128) + iot],
                            v)
                return _
            lax.fori_loop(0, 32, nbody, 0)

        for c in load(0, 0):
            c.start()
        for c in load(1, 1):
            c.start()

        def step(i, b, do_wait_out, do_load_next):
            for c in load(i, b):
                c.wait()
            if do_wait_out:
                store(i - 2, b).wait()
            transpose(b)
            if do_load_next:
                for c in load(i + 2, b):
                    c.start()
            store(i, b).start()

        step(0, 0, False, True)
        step(1, 1, False, True)

        def body(g, carry):
            for b in range(2):
                i = 2 * g + b
                step(i, b, True, True)
            return carry

        lax.fori_loop(1, _BPT // 2 - 1, body, 0)

        step(_BPT - 2, 0, True, False)
        step(_BPT - 1, 1, True, False)
        store(_BPT - 2, 0).wait()
        store(_BPT - 1, 1).wait()

        @pl.when(wid < _REM)
        def _():
            blk = _FULL - _REM + wid
            for tr in range(8):
                pltpu.sync_copy(
                    tt_hbm.at[pl.ds(tr * 8, 8), pl.ds(blk * 128, 128)],
                    inb.at[0, tr])
            transpose(0)
            pltpu.sync_copy(outb.at[0], out_hbm.at[pl.ds(blk * 8, 8)])


    return ka(table_t)


@jax.jit
def _gather_blocks(table_lin, idx_j, tail64):
    @functools.partial(
        pl.kernel,
        mesh=_mesh(),
        out_type=jax.ShapeDtypeStruct((_S, 8, _N // 128, 8, 128), jnp.float32),
        scratch_types=[
            pltpu.VMEM((_JPT * 128,), jnp.int32),
            pltpu.VMEM((_TAILN, _D), jnp.float32),
            pltpu.VMEM((2, 128, _D), jnp.float32),
            pltpu.VMEM((2, 8, 8, 128), jnp.float32),
            pltpu.SemaphoreType.DMA,
            pltpu.SemaphoreType.DMA,
            pltpu.SemaphoreType.DMA,
            pltpu.SemaphoreType.DMA,
        ],
        compiler_params=pltpu.CompilerParams(use_tc_tiling_on_sc=False, needs_layout_passes=False),
    )
    def kb(tab_hbm, idx_hbm, tail_hbm, out_hbm, idxv, tailv, rows, trans,
           g0, g1, w0, w1):
        wid = lax.axis_index("s") * _NC + lax.axis_index("c")
        gsem = (g0, g1)
        wsem = (w0, w1)
        iot = _iota16()
        nvec = [iot + 16 * g for g in range(8)]
        jb0 = wid * _JPT
        tail_lo = _FULL * 128

        pltpu.sync_copy(idx_hbm.at[pl.ds(jb0 * 128, _JPT * 128)], idxv)
        pltpu.sync_copy(tail_hbm, tailv)

        def fixup(i, b):
            accs = []
            masks = []
            tidxs = []
            for g in range(8):
                iv = idxv[pl.ds(i * 128 + 16 * g, 16)]
                m = iv >= tail_lo
                masks.append(m)
                tidxs.append(jnp.maximum(iv - tail_lo, 0))
                accs.append(m.astype(jnp.int32))
            acc = accs[0]
            for a in accs[1:]:
                acc = acc + a
            nhit = lax.reduce_max(acc, axes=(0,))

            @pl.when(nhit > 0)
            def _():
                def dbody(d4, _):
                    for u in range(4):
                        d = d4 * 4 + u
                        dsp = _splat(d)
                        for g in range(8):
                            v = plsc.load_gather(
                                tailv, [tidxs[g], dsp], mask=masks[g])
                            plsc.store_scatter(
                                rows.at[b], [nvec[g], dsp], v,
                                mask=masks[g])
                    return _
                lax.fori_loop(0, 16, dbody, 0)

        def gather(i, b):
            return pltpu.make_async_copy(
                tab_hbm.at[idxv.at[pl.ds(i * 128, 128)]], rows.at[b], gsem[b])

        def store(i, b):
            jb = jb0 + i
            s = jb // (_N // 128)
            tc = jb % (_N // 128)
            return pltpu.make_async_copy(
                trans.at[b], out_hbm.at[s, :, tc], wsem[b])

        def transpose(b):
            def dbody(d4, _):
                for u in range(4):
                    d = d4 * 4 + u
                    dsp = _splat(d)
                    trs = _splat(d // 8)
                    rs = _splat(d % 8)
                    for g in range(8):
                        v = plsc.load_gather(rows.at[b], [nvec[g], dsp])
                        plsc.store_scatter(
                            trans.at[b], [trs, rs, nvec[g]], v)
                return _
            lax.fori_loop(0, 16, dbody, 0)

        gather(0, 0).start()
        gather(1, 1).start()

        def step(i, b, do_wait_out, do_load_next):
            gather(i, b).wait()
            fixup(i, b)
            if do_wait_out:
                store(i - 2, b).wait()
            transpose(b)
            if do_load_next:
                gather(i + 2, b).start()
            store(i, b).start()

        step(0, 0, False, True)
        step(1, 1, False, True)

        def body(g, carry):
            for b in range(2):
                i = 2 * g + b
                step(i, b, True, True)
            return carry

        lax.fori_loop(1, _JPT // 2 - 1, body, 0)

        step(_JPT - 2, 0, True, False)
        step(_JPT - 1, 1, True, False)
        store(_JPT - 2, 0).wait()
        store(_JPT - 1, 1).wait()

    return kb(table_lin, idx_j, tail64)


def kernel(inputs, table):
    table_t = table.T
    tab_lin = _detranspose(table_t).reshape(_V, _D)
    idx_j = inputs.T.reshape(_B).astype(jnp.int32)
    tail64 = table[_FULL * 128:]
    out5 = _gather_blocks(tab_lin, idx_j, tail64)
    return out5.transpose(2, 4, 0, 1, 3).reshape(_N, _S, _D)


# speedup vs baseline: 2.9295x; 2.9295x over previous
import functools

import jax
import jax.numpy as jnp
from jax import lax
from jax.experimental import pallas as pl
from jax.experimental.pallas import tpu as pltpu
from jax.experimental.pallas import tpu_sc as plsc

_D = 64
_NC, _NS = 2, 16
_NW = _NC * _NS
_V = 1000000
_FULL = _V // 128
_BPT = _FULL // _NW
_REM = _FULL - _BPT * _NW
_TAILN = _V - _FULL * 128

_N, _S = 4096, 200
_B = _N * _S
_JBLK = _B // 128
_JPT = _JBLK // _NW


def _mesh():
    return plsc.VectorSubcoreMesh(core_axis_name="c", subcore_axis_name="s")


def _iota16():
    return lax.iota(jnp.int32, 16)


def _splat(x):
    return jnp.full((16,), x, jnp.int32)


@jax.jit
def _detranspose(table_t):
    @functools.partial(
        pl.kernel,
        mesh=_mesh(),
        out_type=jax.ShapeDtypeStruct((_V // 2, 128), jnp.float32),
        scratch_types=[
            pltpu.VMEM((2, 8, 8, 128), jnp.float32),
            pltpu.VMEM((2, 64, 128), jnp.float32),
            pltpu.SemaphoreType.DMA,
            pltpu.SemaphoreType.DMA,
            pltpu.SemaphoreType.DMA,
            pltpu.SemaphoreType.DMA,
        ],
        compiler_params=pltpu.CompilerParams(use_tc_tiling_on_sc=True, needs_layout_passes=False),
    )
    def ka(tt_hbm, out_hbm, inb, outb, gi0, gi1, wo0, wo1):
        wid = lax.axis_index("s") * _NC + lax.axis_index("c")
        gsem = (gi0, gi1)
        wsem = (wo0, wo1)
        iot = _iota16()
        dtr = [(iot + d0) >> 3 for d0 in (0, 16, 32, 48)]
        ddr = [(iot + d0) & 7 for d0 in (0, 16, 32, 48)]

        def blk_of(i):
            return i * _NW + wid

        def load(i, b):
            blk = blk_of(i)
            return [
                pltpu.make_async_copy(
                    tt_hbm.at[pl.ds(tr * 8, 8), pl.ds(blk * 128, 128)],
                    inb.at[b, tr], gsem[b])
                for tr in range(8)
            ]

        def store(i, b):
            blk = blk_of(i)
            return pltpu.make_async_copy(
                outb.at[b], out_hbm.at[pl.ds(blk * 64, 64)], wsem[b])

        def transpose(b):
            inb_b = inb.at[b]
            outb_b = outb.at[b]

            def kbody(k, _):
                nmod = (iot + k) & 15
                nhalf = nmod >> 1
                qrel = ((nmod & 1) << 6) + iot
                for di in range(4):
                    qv = qrel + 16 * di
                    for g in range(8):
                        n0 = 16 * g
                        nidx = nmod + n0
                        pv = nhalf + (n0 >> 1)
                        v = plsc.load_gather(inb_b, [dtr[di], ddr[di], nidx])
                        plsc.store_scatter(outb_b, [pv, qv], v)
                return _
            lax.fori_loop(0, 16, kbody, 0)

        for c in load(0, 0):
            c.start()
        for c in load(1, 1):
            c.start()

        def step(i, b, do_wait_out, do_load_next):
            for c in load(i, b):
                c.wait()
            if do_wait_out:
                store(i - 2, b).wait()
            transpose(b)
            if do_load_next:
                for c in load(i + 2, b):
                    c.start()
            store(i, b).start()

        step(0, 0, False, True)
        step(1, 1, False, True)

        def body(g, carry):
            for b in range(2):
                i = 2 * g + b
                step(i, b, True, True)
            return carry

        lax.fori_loop(1, _BPT // 2 - 1, body, 0)

        step(_BPT - 2, 0, True, False)
        step(_BPT - 1, 1, True, False)
        store(_BPT - 2, 0).wait()
        store(_BPT - 1, 1).wait()

        @pl.when(wid < _REM)
        def _():
            blk = _FULL - _REM + wid
            for tr in range(8):
                pltpu.sync_copy(
                    tt_hbm.at[pl.ds(tr * 8, 8), pl.ds(blk * 128, 128)],
                    inb.at[0, tr])
            transpose(0)
            pltpu.sync_copy(outb.at[0], out_hbm.at[pl.ds(blk * 64, 64)])


    return ka(table_t)


@jax.jit
def _gather_blocks(table_lin, idx_j, tail64):
    @functools.partial(
        pl.kernel,
        mesh=_mesh(),
        out_type=jax.ShapeDtypeStruct((_S, 8, _N // 128, 8, 128), jnp.float32),
        scratch_types=[
            pltpu.VMEM((_JPT * 128,), jnp.int32),
            pltpu.VMEM((_TAILN, _D), jnp.float32),
            pltpu.VMEM((2, 128, _D), jnp.float32),
            pltpu.VMEM((2, 8, 8, 128), jnp.float32),
            pltpu.SemaphoreType.DMA,
            pltpu.SemaphoreType.DMA,
            pltpu.SemaphoreType.DMA,
            pltpu.SemaphoreType.DMA,
        ],
        compiler_params=pltpu.CompilerParams(use_tc_tiling_on_sc=False, needs_layout_passes=False),
    )
    def kb(tab_hbm, idx_hbm, tail_hbm, out_hbm, idxv, tailv, rows, trans,
           g0, g1, w0, w1):
        wid = lax.axis_index("s") * _NC + lax.axis_index("c")
        gsem = (g0, g1)
        wsem = (w0, w1)
        iot = _iota16()
        nvec = [iot + 16 * g for g in range(8)]
        jb0 = wid * _JPT
        tail_lo = _FULL * 128

        pltpu.sync_copy(idx_hbm.at[pl.ds(jb0 * 128, _JPT * 128)], idxv)
        pltpu.sync_copy(tail_hbm, tailv)

        def fixup(i, b):
            accs = []
            masks = []
            tidxs = []
            for g in range(8):
                iv = idxv[pl.ds(i * 128 + 16 * g, 16)]
                m = iv >= tail_lo
                masks.append(m)
                tidxs.append(jnp.maximum(iv - tail_lo, 0))
                accs.append(m.astype(jnp.int32))
            acc = accs[0]
            for a in accs[1:]:
                acc = acc + a
            nhit = lax.reduce_max(acc, axes=(0,))

            @pl.when(nhit > 0)
            def _():
                def dbody(d4, _):
                    for u in range(4):
                        d = d4 * 4 + u
                        dsp = _splat(d)
                        for g in range(8):
                            v = plsc.load_gather(
                                tailv, [tidxs[g], dsp], mask=masks[g])
                            plsc.store_scatter(
                                rows.at[b], [nvec[g], dsp], v,
                                mask=masks[g])
                    return _
                lax.fori_loop(0, 16, dbody, 0)

        def gather(i, b):
            return pltpu.make_async_copy(
                tab_hbm.at[idxv.at[pl.ds(i * 128, 128)]], rows.at[b], gsem[b])

        def store(i, b):
            jb = jb0 + i
            s = jb // (_N // 128)
            tc = jb % (_N // 128)
            return pltpu.make_async_copy(
                trans.at[b], out_hbm.at[s, :, tc], wsem[b])

        def transpose(b):
            rows_b = rows.at[b]
            trans_b = trans.at[b]

            def kbody(k, _):
                dmod = (iot + k) & 15
                trrel = dmod >> 3
                rrel = dmod & 7
                for di in range(4):
                    d0 = 16 * di
                    didx = dmod + d0
                    trv = trrel + (d0 >> 3)
                    for g in range(8):
                        v = plsc.load_gather(rows_b, [nvec[g], didx])
                        plsc.store_scatter(trans_b, [trv, rrel, nvec[g]], v)
                return _
            lax.fori_loop(0, 16, kbody, 0)

        gather(0, 0).start()
        gather(1, 1).start()

        def step(i, b, do_wait_out, do_load_next):
            gather(i, b).wait()
            fixup(i, b)
            if do_wait_out:
                store(i - 2, b).wait()
            transpose(b)
            if do_load_next:
                gather(i + 2, b).start()
            store(i, b).start()

        step(0, 0, False, True)
        step(1, 1, False, True)

        def body(g, carry):
            for b in range(2):
                i = 2 * g + b
                step(i, b, True, True)
            return carry

        lax.fori_loop(1, _JPT // 2 - 1, body, 0)

        step(_JPT - 2, 0, True, False)
        step(_JPT - 1, 1, True, False)
        store(_JPT - 2, 0).wait()
        store(_JPT - 1, 1).wait()

    return kb(table_lin, idx_j, tail64)


def kernel(inputs, table):
    table_t = table.T
    tab_lin = _detranspose(table_t).reshape(_V, _D)
    idx_j = inputs.T.reshape(_B).astype(jnp.int32)
    tail64 = table[_FULL * 128:]
    out5 = _gather_blocks(tab_lin, idx_j, tail64)
    return out5.transpose(2, 4, 0, 1, 3).reshape(_N, _S, _D)
